# baseline (speedup 1.0000x reference)
import functools

import jax
import jax.numpy as jnp
from jax import lax
from jax.experimental import pallas as pl
from jax.experimental.pallas import tpu as pltpu
from jax.experimental.pallas import tpu_sc as plsc

N = 10000
D = 128
C = 128
E = 320000
NEG_SLOPE = 0.2

NC = 2
NS = 16
NW = NC * NS
B = 128
CHUNKS = 82
EW = CHUNKS * B
EP = NW * EW
CW = C + 16
NP = 10016
RT = NP // NS

_SC_PARAMS = pltpu.CompilerParams(needs_layout_passes=False,
                                  use_tc_tiling_on_sc=False)


def _mm_body(x_ref, wl_ref, wr_ref, xl_ref, xr_ref):
    xb = x_ref[...]
    xl_ref[...] = jnp.dot(xb, wl_ref[...], preferred_element_type=jnp.float32)
    xr_ref[...] = jnp.dot(xb, wr_ref[...], preferred_element_type=jnp.float32)


def _project(x, W_l, W_r):
    blk = 1000
    return pl.pallas_call(
        _mm_body,
        grid=(N // blk,),
        in_specs=[
            pl.BlockSpec((blk, D), lambda i: (i, 0)),
            pl.BlockSpec((D, C), lambda i: (0, 0)),
            pl.BlockSpec((D, C), lambda i: (0, 0)),
        ],
        out_specs=[
            pl.BlockSpec((blk, C), lambda i: (i, 0)),
            pl.BlockSpec((blk, C), lambda i: (i, 0)),
        ],
        out_shape=[
            jax.ShapeDtypeStruct((N, C), jnp.float32),
            jax.ShapeDtypeStruct((N, C), jnp.float32),
        ],
    )(x, W_l, W_r)


def _logits_body(xlp_hbm, xr_hbm, src_hbm, dstg_hbm, att_hbm,
                 e_hbm, tmax_hbm,
                 att_v, src_w, dst_w, e_w, mx_v, ts_v,
                 rows_l, rows_r, sem_l, sem_r):
    c = lax.axis_index("c")
    s = lax.axis_index("s")
    wid = s * NC + c
    base = wid * EW

    pltpu.sync_copy(att_hbm, att_v)
    pltpu.sync_copy(src_hbm.at[pl.ds(wid * CHUNKS, CHUNKS)], src_w)
    pltpu.sync_copy(dstg_hbm.at[pl.ds(wid * CHUNKS, CHUNKS)], dst_w)
    att_regs = [att_v[pl.ds(j * 16, 16)] for j in range(C // 16)]
    iota = lax.iota(jnp.int32, 16)

    def start_gather(t, b):
        pltpu.make_async_copy(xlp_hbm.at[src_w.at[t]], rows_l[b], sem_l[b]).start()
        pltpu.make_async_copy(xr_hbm.at[dst_w.at[t]], rows_r[b], sem_r[b]).start()

    def wait_gather(t, b):
        pltpu.make_async_copy(xlp_hbm.at[src_w.at[t]], rows_l[b], sem_l[b]).wait()
        pltpu.make_async_copy(xr_hbm.at[dst_w.at[t]], rows_r[b], sem_r[b]).wait()

    start_gather(0, 0)

    def pair(t2, mx):
        for b in range(2):
            t = t2 * 2 + b

            @pl.when(t + 1 < CHUNKS)
            def _():
                start_gather(t + 1, 1 - b)

            wait_gather(t, b)
            rl, rr = rows_l[b], rows_r[b]

            def egroup(g, carry):
                for l in range(16):
                    i = g * 16 + l
                    acc = jnp.zeros((16,), jnp.float32)
                    for j in range(C // 16):
                        m = (rl[i, pl.ds(j * 16, 16)]
                             + rr[i, pl.ds(j * 16, 16)])
                        m = jnp.maximum(m, NEG_SLOPE * m)
                        acc = acc + att_regs[j] * m
                    ts_v[pl.ds(l * 16, 16)] = acc
                tot = jnp.zeros((16,), jnp.float32)
                iota16 = iota * 16
                for cc in range(16):
                    tot = tot + plsc.load_gather(ts_v, [iota16 + cc])
                e_w[pl.ds(t * B + g * 16, 16)] = tot
                return jnp.maximum(carry, tot)

            mx = lax.fori_loop(0, B // 16, egroup, mx)
        return mx

    mx = lax.fori_loop(0, CHUNKS // 2, pair,
                       jnp.full((16,), -1e30, jnp.float32))
    pltpu.sync_copy(e_w, e_hbm.at[pl.ds(base, EW)])
    mx_v[...] = mx
    pltpu.sync_copy(mx_v, tmax_hbm.at[wid])


def _logits(xlp, xr, src2, dstg2, att):
    mesh = plsc.VectorSubcoreMesh(core_axis_name="c", subcore_axis_name="s",
                                  num_cores=NC, num_subcores=NS)
    return pl.kernel(
        _logits_body,
        out_type=(
            jax.ShapeDtypeStruct((EP,), jnp.float32),
            jax.ShapeDtypeStruct((NW, 16), jnp.float32),
        ),
        mesh=mesh,
        scratch_types=[
            pltpu.VMEM((C,), jnp.float32),
            pltpu.VMEM((CHUNKS, B), jnp.int32),
            pltpu.VMEM((CHUNKS, B), jnp.int32),
            pltpu.VMEM((EW,), jnp.float32),
            pltpu.VMEM((16,), jnp.float32),
            pltpu.VMEM((256,), jnp.float32),
            [pltpu.VMEM((B, CW), jnp.float32) for _ in range(2)],
            [pltpu.VMEM((B, C), jnp.float32) for _ in range(2)],
            [pltpu.SemaphoreType.DMA for _ in range(2)],
            [pltpu.SemaphoreType.DMA for _ in range(2)],
        ],
        compiler_params=_SC_PARAMS,
    )(xlp, xr, src2, dstg2, att)


def _aggregate_body(xlp_hbm, idxe_hbm, tmax_hbm, zeros_hbm,
                    accs_hbm,
                    idxe_v, dst_cur, w_v, tm_v, acc_sh,
                    rows_v, sem_i, sem_g, sem_s):
    c = lax.axis_index("c")
    s = lax.axis_index("s")
    wid = s * NC + c
    crow = wid * CHUNKS

    pltpu.sync_copy(zeros_hbm.at[pl.ds(s * RT, RT)],
                    acc_sh.at[pl.ds(s * RT, RT)])

    pltpu.sync_copy(tmax_hbm, tm_v)
    gmv = jnp.full((16,), -1e30, jnp.float32)
    for r in range(NW):
        gmv = jnp.maximum(gmv, tm_v[r])
    gmax = jnp.max(gmv)

    plsc.subcore_barrier()

    def idxe_copy(t, b):
        return pltpu.make_async_copy(idxe_hbm.at[crow + t], idxe_v.at[b],
                                     sem_i[b])

    def gather(t, b):
        return pltpu.make_async_copy(xlp_hbm.at[idxe_v.at[b, 0]], rows_v[b],
                                     sem_g[b])

    def scatter(b):
        return pltpu.make_async_copy(rows_v[b], acc_sh.at[dst_cur[b]],
                                     sem_s[b])

    idxe_copy(0, 0).start()
    idxe_copy(0, 0).wait()
    idxe_copy(1, 1).start()
    gather(0, 0).start()

    def pair(t2, carry):
        for b in range(2):
            t = t2 * 2 + b

            @pl.when(t >= 1)
            def _():
                scatter(1 - b).wait()

            @pl.when(t + 1 < CHUNKS)
            def _():
                idxe_copy(t + 1, 1 - b).wait()
                gather(t + 1, 1 - b).start()

            gather(t, b).wait()

            def pregroup(g, icarry):
                ds16 = pl.ds(g * 16, 16)
                ev = plsc.bitcast(idxe_v[b, 2, ds16], jnp.float32)
                w_v[ds16] = jnp.exp(ev - gmax)
                dst_cur[b][ds16] = idxe_v[b, 1, ds16]
                return icarry

            lax.fori_loop(0, B // 16, pregroup, 0)

            @pl.when(t + 2 < CHUNKS)
            def _():
                idxe_copy(t + 2, b).start()

            rv = rows_v[b]

            def egroup(g, icarry):
                for l in range(16):
                    i = g * 16 + l
                    bw = plsc.load_gather(
                        w_v, [jnp.full((16,), i, jnp.int32)])
                    for j in range(CW // 16):
                        rv[i, pl.ds(j * 16, 16)] = rv[i, pl.ds(j * 16, 16)] * bw
                return icarry

            lax.fori_loop(0, B // 16, egroup, 0)
            scatter(b).start(add=True)
        return carry

    lax.fori_loop(0, CHUNKS // 2, pair, 0)
    scatter((CHUNKS - 1) % 2).wait()
    plsc.subcore_barrier()
    pltpu.sync_copy(acc_sh.at[pl.ds(s * RT, RT)],
                    accs_hbm.at[c, pl.ds(s * RT, RT)])


def _aggregate(xlp, idxe, tmax, zeros_acc):
    mesh = plsc.VectorSubcoreMesh(core_axis_name="c", subcore_axis_name="s",
                                  num_cores=NC, num_subcores=NS)
    return pl.kernel(
        _aggregate_body,
        out_type=jax.ShapeDtypeStruct((NC, NP, CW), jnp.float32),
        mesh=mesh,
        scratch_types=[
            pltpu.VMEM((2, 3, B), jnp.int32),
            [pltpu.VMEM((B,), jnp.int32) for _ in range(2)],
            pltpu.VMEM((B,), jnp.float32),
            pltpu.VMEM((NW, 16), jnp.float32),
            pltpu.VMEM_SHARED((NP, CW), jnp.float32),
            [pltpu.VMEM((B, CW), jnp.float32) for _ in range(2)],
            [pltpu.SemaphoreType.DMA for _ in range(2)],
            [pltpu.SemaphoreType.DMA for _ in range(2)],
            [pltpu.SemaphoreType.DMA for _ in range(2)],
        ],
        compiler_params=_SC_PARAMS,
    )(xlp, idxe, tmax, zeros_acc)


def _norm_body(accs_ref, bias_ref, gamma_ref, beta_ref, agn_ref, out_ref):
    a = accs_ref[0] + accs_ref[1]
    col = lax.broadcasted_iota(jnp.int32, (NP, CW), 1)
    den_full = jnp.where(col == C, a, 0.0)
    den = jnp.sum(den_full, axis=1, keepdims=True)
    num = a[:N, :C]
    out0 = num / (den[:N] + 1e-16) + bias_ref[...]
    mean = jnp.mean(out0, axis=0, keepdims=True)
    out_c = out0 - agn_ref[...] * mean
    var = jnp.mean(out_c * out_c, axis=0, keepdims=True)
    out_ref[...] = gamma_ref[...] * out_c / jnp.sqrt(var + 1e-5) + beta_ref[...]


def _finalize(accs, bias, gamma, beta, alpha_gn):
    return pl.pallas_call(
        _norm_body,
        out_shape=jax.ShapeDtypeStruct((N, C), jnp.float32),
    )(accs, bias.reshape(1, C), gamma.reshape(1, C), beta.reshape(1, C),
      alpha_gn.reshape(1, C))


@jax.jit
def kernel(x, edge_index, W_l, W_r, att, bias, gamma, beta, alpha_gn):
    ei = edge_index.astype(jnp.int32)
    loop = jnp.arange(N, dtype=jnp.int32)
    pad = EP - (E + N)
    src = jnp.concatenate([ei[0], loop, jnp.zeros((pad,), jnp.int32)])
    dstg = jnp.concatenate([ei[1], loop, jnp.zeros((pad,), jnp.int32)])
    dsts = jnp.concatenate([ei[1], loop, jnp.full((pad,), N, jnp.int32)])
    src2 = src.reshape(NW * CHUNKS, B)
    dstg2 = dstg.reshape(NW * CHUNKS, B)
    dsts2 = dsts.reshape(NW * CHUNKS, B)

    xl, xr = _project(x, W_l, W_r)
    extra = jnp.concatenate(
        [jnp.ones((N, 1), jnp.float32), jnp.zeros((N, 15), jnp.float32)], axis=1)
    xlp = jnp.concatenate([xl, extra], axis=1)

    e, tmax = _logits(xlp, xr, src2, dstg2, att.reshape(C))
    e_i32 = lax.bitcast_convert_type(e, jnp.int32)
    idxe = (jnp.stack([src, dsts, e_i32], axis=0)
            .reshape(3, NW * CHUNKS, B).transpose(1, 0, 2))
    zeros_acc = jnp.zeros((NP, CW), jnp.float32)
    accs = _aggregate(xlp, idxe, tmax, zeros_acc)
    return _finalize(accs, bias, gamma, beta, alpha_gn)


# speedup vs baseline: 1.1519x; 1.1519x over previous
import functools

import jax
import jax.numpy as jnp
from jax import lax
from jax.experimental import pallas as pl
from jax.experimental.pallas import tpu as pltpu
from jax.experimental.pallas import tpu_sc as plsc

N = 10000
D = 128
C = 128
E = 320000
NEG_SLOPE = 0.2

NC = 2
NS = 16
NW = NC * NS
B = 128
CHUNKS = 82
EW = CHUNKS * B
EP = NW * EW
CH_SUM = 2 * CHUNKS
CH0 = 114
CH1 = CH_SUM - CH0
CW = C + 16
NP = 10016
RT = NP // NS

_SC_PARAMS = pltpu.CompilerParams(needs_layout_passes=False,
                                  use_tc_tiling_on_sc=False)


def _mm_body(x_ref, wl_ref, wr_ref, xl_ref, xr_ref):
    xb = x_ref[...]
    xl_ref[...] = jnp.dot(xb, wl_ref[...], preferred_element_type=jnp.float32)
    xr_ref[...] = jnp.dot(xb, wr_ref[...], preferred_element_type=jnp.float32)


def _project(x, W_l, W_r):
    blk = 1000
    return pl.pallas_call(
        _mm_body,
        grid=(N // blk,),
        in_specs=[
            pl.BlockSpec((blk, D), lambda i: (i, 0)),
            pl.BlockSpec((D, C), lambda i: (0, 0)),
            pl.BlockSpec((D, C), lambda i: (0, 0)),
        ],
        out_specs=[
            pl.BlockSpec((blk, C), lambda i: (i, 0)),
            pl.BlockSpec((blk, C), lambda i: (i, 0)),
        ],
        out_shape=[
            jax.ShapeDtypeStruct((N, C), jnp.float32),
            jax.ShapeDtypeStruct((N, C), jnp.float32),
        ],
    )(x, W_l, W_r)


def _logits_body(xlp_hbm, xr_hbm, src_hbm, dstg_hbm, att_hbm,
                 e_hbm, tmax_hbm,
                 att_v, src_w, dst_w, e_w, mx_v, ts_v,
                 rows_l, rows_r, sem_l, sem_r):
    c = lax.axis_index("c")
    s = lax.axis_index("s")
    wid = s * NC + c
    crow = s * CH_SUM + c * CH0
    nch = jnp.where(c == 0, CH0, CH1)

    pltpu.sync_copy(att_hbm, att_v)

    @pl.when(c == 0)
    def _():
        pltpu.sync_copy(src_hbm.at[pl.ds(crow, CH0)], src_w)
        pltpu.sync_copy(dstg_hbm.at[pl.ds(crow, CH0)], dst_w)

    @pl.when(c == 1)
    def _():
        pltpu.sync_copy(src_hbm.at[pl.ds(crow, CH1)], src_w.at[pl.ds(0, CH1)])
        pltpu.sync_copy(dstg_hbm.at[pl.ds(crow, CH1)], dst_w.at[pl.ds(0, CH1)])

    att_regs = [att_v[pl.ds(j * 16, 16)] for j in range(C // 16)]
    iota = lax.iota(jnp.int32, 16)

    def start_gather(t, b):
        pltpu.make_async_copy(xlp_hbm.at[src_w.at[t]], rows_l[b], sem_l[b]).start()
        pltpu.make_async_copy(xr_hbm.at[dst_w.at[t]], rows_r[b], sem_r[b]).start()

    def wait_gather(t, b):
        pltpu.make_async_copy(xlp_hbm.at[src_w.at[t]], rows_l[b], sem_l[b]).wait()
        pltpu.make_async_copy(xr_hbm.at[dst_w.at[t]], rows_r[b], sem_r[b]).wait()

    start_gather(0, 0)

    def pair(t2, mx):
        for b in range(2):
            t = t2 * 2 + b

            @pl.when(t + 1 < nch)
            def _():
                start_gather(t + 1, 1 - b)

            wait_gather(t, b)
            rl, rr = rows_l[b], rows_r[b]

            def egroup(g, carry):
                for l in range(16):
                    i = g * 16 + l
                    acc = jnp.zeros((16,), jnp.float32)
                    for j in range(C // 16):
                        m = (rl[i, pl.ds(j * 16, 16)]
                             + rr[i, pl.ds(j * 16, 16)])
                        m = jnp.maximum(m, NEG_SLOPE * m)
                        acc = acc + att_regs[j] * m
                    ts_v[pl.ds(l * 16, 16)] = acc
                tot = jnp.zeros((16,), jnp.float32)
                iota16 = iota * 16
                for cc in range(16):
                    tot = tot + plsc.load_gather(ts_v, [iota16 + cc])
                e_w[pl.ds(t * B + g * 16, 16)] = tot
                return jnp.maximum(carry, tot)

            mx = lax.fori_loop(0, B // 16, egroup, mx)
        return mx

    mx = lax.fori_loop(0, nch // 2, pair,
                       jnp.full((16,), -1e30, jnp.float32))

    @pl.when(c == 0)
    def _():
        pltpu.sync_copy(e_w, e_hbm.at[pl.ds(crow * B, CH0 * B)])

    @pl.when(c == 1)
    def _():
        pltpu.sync_copy(e_w.at[pl.ds(0, CH1 * B)],
                        e_hbm.at[pl.ds(crow * B, CH1 * B)])

    mx_v[...] = mx
    pltpu.sync_copy(mx_v, tmax_hbm.at[wid])


def _logits(xlp, xr, src2, dstg2, att):
    mesh = plsc.VectorSubcoreMesh(core_axis_name="c", subcore_axis_name="s",
                                  num_cores=NC, num_subcores=NS)
    return pl.kernel(
        _logits_body,
        out_type=(
            jax.ShapeDtypeStruct((EP,), jnp.float32),
            jax.ShapeDtypeStruct((NW, 16), jnp.float32),
        ),
        mesh=mesh,
        scratch_types=[
            pltpu.VMEM((C,), jnp.float32),
            pltpu.VMEM((CH0, B), jnp.int32),
            pltpu.VMEM((CH0, B), jnp.int32),
            pltpu.VMEM((CH0 * B,), jnp.float32),
            pltpu.VMEM((16,), jnp.float32),
            pltpu.VMEM((256,), jnp.float32),
            [pltpu.VMEM((B, CW), jnp.float32) for _ in range(2)],
            [pltpu.VMEM((B, C), jnp.float32) for _ in range(2)],
            [pltpu.SemaphoreType.DMA for _ in range(2)],
            [pltpu.SemaphoreType.DMA for _ in range(2)],
        ],
        compiler_params=_SC_PARAMS,
    )(xlp, xr, src2, dstg2, att)


def _aggregate_body(xlp_hbm, idxe_hbm, tmax_hbm, zeros_hbm,
                    accs_hbm,
                    idxe_v, dst_cur, w_v, tm_v, acc_sh,
                    rows_v, sem_i, sem_g, sem_s):
    c = lax.axis_index("c")
    s = lax.axis_index("s")
    crow = s * CH_SUM + c * CH0
    nch = jnp.where(c == 0, CH0, CH1)

    pltpu.sync_copy(zeros_hbm.at[pl.ds(s * RT, RT)],
                    acc_sh.at[pl.ds(s * RT, RT)])

    pltpu.sync_copy(tmax_hbm, tm_v)
    gmv = jnp.full((16,), -1e30, jnp.float32)
    for r in range(NW):
        gmv = jnp.maximum(gmv, tm_v[r])
    gmax = jnp.max(gmv)

    plsc.subcore_barrier()

    def idxe_copy(t, b):
        return pltpu.make_async_copy(idxe_hbm.at[crow + t], idxe_v.at[b],
                                     sem_i[b])

    def gather(t, b):
        return pltpu.make_async_copy(xlp_hbm.at[idxe_v.at[b, 0]], rows_v[b],
                                     sem_g[b])

    def scatter(b):
        return pltpu.make_async_copy(rows_v[b], acc_sh.at[dst_cur[b]],
                                     sem_s[b])

    idxe_copy(0, 0).start()
    idxe_copy(0, 0).wait()
    idxe_copy(1, 1).start()
    gather(0, 0).start()

    def pair(t2, carry):
        for b in range(2):
            t = t2 * 2 + b

            @pl.when(t >= 1)
            def _():
                scatter(1 - b).wait()

            @pl.when(t + 1 < nch)
            def _():
                idxe_copy(t + 1, 1 - b).wait()
                gather(t + 1, 1 - b).start()

            gather(t, b).wait()

            def pregroup(g, icarry):
                ds16 = pl.ds(g * 16, 16)
                ev = plsc.bitcast(idxe_v[b, 2, ds16], jnp.float32)
                w_v[ds16] = jnp.exp(ev - gmax)
                dst_cur[b][ds16] = idxe_v[b, 1, ds16]
                return icarry

            lax.fori_loop(0, B // 16, pregroup, 0)

            @pl.when(t + 2 < nch)
            def _():
                idxe_copy(t + 2, b).start()

            rv = rows_v[b]

            def egroup(g, icarry):
                for l in range(16):
                    i = g * 16 + l
                    bw = plsc.load_gather(
                        w_v, [jnp.full((16,), i, jnp.int32)])
                    for j in range(CW // 16):
                        rv[i, pl.ds(j * 16, 16)] = rv[i, pl.ds(j * 16, 16)] * bw
                return icarry

            lax.fori_loop(0, B // 16, egroup, 0)
            scatter(b).start(add=True)
        return carry

    lax.fori_loop(0, nch // 2, pair, 0)
    scatter(1).wait()
    plsc.subcore_barrier()
    pltpu.sync_copy(acc_sh.at[pl.ds(s * RT, RT)],
                    accs_hbm.at[c, pl.ds(s * RT, RT)])


def _aggregate(xlp, idxe, tmax, zeros_acc):
    mesh = plsc.VectorSubcoreMesh(core_axis_name="c", subcore_axis_name="s",
                                  num_cores=NC, num_subcores=NS)
    return pl.kernel(
        _aggregate_body,
        out_type=jax.ShapeDtypeStruct((NC, NP, CW), jnp.float32),
        mesh=mesh,
        scratch_types=[
            pltpu.VMEM((2, 3, B), jnp.int32),
            [pltpu.VMEM((B,), jnp.int32) for _ in range(2)],
            pltpu.VMEM((B,), jnp.float32),
            pltpu.VMEM((NW, 16), jnp.float32),
            pltpu.VMEM_SHARED((NP, CW), jnp.float32),
            [pltpu.VMEM((B, CW), jnp.float32) for _ in range(2)],
            [pltpu.SemaphoreType.DMA for _ in range(2)],
            [pltpu.SemaphoreType.DMA for _ in range(2)],
            [pltpu.SemaphoreType.DMA for _ in range(2)],
        ],
        compiler_params=_SC_PARAMS,
    )(xlp, idxe, tmax, zeros_acc)


def _norm_body(accs_ref, bias_ref, gamma_ref, beta_ref, agn_ref, out_ref):
    a = accs_ref[0] + accs_ref[1]
    col = lax.broadcasted_iota(jnp.int32, (NP, CW), 1)
    den_full = jnp.where(col == C, a, 0.0)
    den = jnp.sum(den_full, axis=1, keepdims=True)
    num = a[:N, :C]
    out0 = num / (den[:N] + 1e-16) + bias_ref[...]
    mean = jnp.mean(out0, axis=0, keepdims=True)
    out_c = out0 - agn_ref[...] * mean
    var = jnp.mean(out_c * out_c, axis=0, keepdims=True)
    out_ref[...] = gamma_ref[...] * out_c / jnp.sqrt(var + 1e-5) + beta_ref[...]


def _finalize(accs, bias, gamma, beta, alpha_gn):
    return pl.pallas_call(
        _norm_body,
        out_shape=jax.ShapeDtypeStruct((N, C), jnp.float32),
    )(accs, bias.reshape(1, C), gamma.reshape(1, C), beta.reshape(1, C),
      alpha_gn.reshape(1, C))


@jax.jit
def kernel(x, edge_index, W_l, W_r, att, bias, gamma, beta, alpha_gn):
    ei = edge_index.astype(jnp.int32)
    loop = jnp.arange(N, dtype=jnp.int32)
    pad = EP - (E + N)
    src = jnp.concatenate([ei[0], loop, jnp.zeros((pad,), jnp.int32)])
    dstg = jnp.concatenate([ei[1], loop, jnp.zeros((pad,), jnp.int32)])
    dsts = jnp.concatenate([ei[1], loop, jnp.full((pad,), N, jnp.int32)])
    src2 = src.reshape(NW * CHUNKS, B)
    dstg2 = dstg.reshape(NW * CHUNKS, B)

    xl, xr = _project(x, W_l, W_r)
    extra = jnp.concatenate(
        [jnp.ones((N, 1), jnp.float32), jnp.zeros((N, 15), jnp.float32)], axis=1)
    xlp = jnp.concatenate([xl, extra], axis=1)

    e, tmax = _logits(xlp, xr, src2, dstg2, att.reshape(C))
    e_i32 = lax.bitcast_convert_type(e, jnp.int32)
    idxe = (jnp.stack([src, dsts, e_i32], axis=0)
            .reshape(3, NW * CHUNKS, B).transpose(1, 0, 2))
    zeros_acc = jnp.zeros((NP, CW), jnp.float32)
    accs = _aggregate(xlp, idxe, tmax, zeros_acc)
    return _finalize(accs, bias, gamma, beta, alpha_gn)


# speedup vs baseline: 1.1539x; 1.0017x over previous
import functools

import jax
import jax.numpy as jnp
from jax import lax
from jax.experimental import pallas as pl
from jax.experimental.pallas import tpu as pltpu
from jax.experimental.pallas import tpu_sc as plsc

N = 10000
D = 128
C = 128
E = 320000
NEG_SLOPE = 0.2

NC = 2
NS = 16
NW = NC * NS
B = 128
CHUNKS = 82
EW = CHUNKS * B
EP = NW * EW
CH_SUM = 2 * CHUNKS
CH0 = 114
CH1 = CH_SUM - CH0
CW = C + 16
NP = 10016
RT = NP // NS

_SC_PARAMS = pltpu.CompilerParams(needs_layout_passes=False,
                                  use_tc_tiling_on_sc=False)


def _mm_body(x_ref, wl_ref, wr_ref, xl_ref, xr_ref):
    xb = x_ref[...]
    xl_ref[...] = jnp.dot(xb, wl_ref[...], preferred_element_type=jnp.float32)
    xr_ref[...] = jnp.dot(xb, wr_ref[...], preferred_element_type=jnp.float32)


def _project(x, W_l, W_r):
    blk = 1000
    return pl.pallas_call(
        _mm_body,
        grid=(N // blk,),
        in_specs=[
            pl.BlockSpec((blk, D), lambda i: (i, 0)),
            pl.BlockSpec((D, C), lambda i: (0, 0)),
            pl.BlockSpec((D, C), lambda i: (0, 0)),
        ],
        out_specs=[
            pl.BlockSpec((blk, C), lambda i: (i, 0)),
            pl.BlockSpec((blk, C), lambda i: (i, 0)),
        ],
        out_shape=[
            jax.ShapeDtypeStruct((N, C), jnp.float32),
            jax.ShapeDtypeStruct((N, C), jnp.float32),
        ],
    )(x, W_l, W_r)


def _logits_body(xlp_hbm, xr_hbm, src_hbm, dstg_hbm, att_hbm,
                 e_hbm, tmax_hbm,
                 att_v, src_w, dst_w, e_w, mx_v, ts_v,
                 rows_l, rows_r, sem_l, sem_r):
    c = lax.axis_index("c")
    s = lax.axis_index("s")
    wid = s * NC + c
    crow = s * CH_SUM + c * CH0
    nch = jnp.where(c == 0, CH0, CH1)

    pltpu.sync_copy(att_hbm, att_v)

    @pl.when(c == 0)
    def _():
        pltpu.sync_copy(src_hbm.at[pl.ds(crow, CH0)], src_w)
        pltpu.sync_copy(dstg_hbm.at[pl.ds(crow, CH0)], dst_w)

    @pl.when(c == 1)
    def _():
        pltpu.sync_copy(src_hbm.at[pl.ds(crow, CH1)], src_w.at[pl.ds(0, CH1)])
        pltpu.sync_copy(dstg_hbm.at[pl.ds(crow, CH1)], dst_w.at[pl.ds(0, CH1)])

    att_regs = [att_v[pl.ds(j * 16, 16)] for j in range(C // 16)]
    iota = lax.iota(jnp.int32, 16)

    def start_gather(t, b):
        pltpu.make_async_copy(xlp_hbm.at[src_w.at[t]], rows_l[b], sem_l[b]).start()
        pltpu.make_async_copy(xr_hbm.at[dst_w.at[t]], rows_r[b], sem_r[b]).start()

    def wait_gather(t, b):
        pltpu.make_async_copy(xlp_hbm.at[src_w.at[t]], rows_l[b], sem_l[b]).wait()
        pltpu.make_async_copy(xr_hbm.at[dst_w.at[t]], rows_r[b], sem_r[b]).wait()

    start_gather(0, 0)

    def pair(t2, mx):
        for b in range(2):
            t = t2 * 2 + b

            @pl.when(t + 1 < nch)
            def _():
                start_gather(t + 1, 1 - b)

            wait_gather(t, b)
            rl, rr = rows_l[b], rows_r[b]

            def egroup(g, carry):
                def lane(l, lcarry):
                    i = g * 16 + l
                    acc = jnp.zeros((16,), jnp.float32)
                    for j in range(C // 16):
                        m = (rl[i, pl.ds(j * 16, 16)]
                             + rr[i, pl.ds(j * 16, 16)])
                        m = jnp.maximum(m, NEG_SLOPE * m)
                        acc = acc + att_regs[j] * m
                    ts_v[pl.ds(l * 16, 16)] = acc
                    return lcarry

                lax.fori_loop(0, 16, lane, 0)
                tot = jnp.zeros((16,), jnp.float32)
                iota16 = iota * 16
                for cc in range(16):
                    tot = tot + plsc.load_gather(ts_v, [iota16 + cc])
                e_w[pl.ds(t * B + g * 16, 16)] = tot
                return jnp.maximum(carry, tot)

            mx = lax.fori_loop(0, B // 16, egroup, mx)
        return mx

    mx = lax.fori_loop(0, nch // 2, pair,
                       jnp.full((16,), -1e30, jnp.float32))

    @pl.when(c == 0)
    def _():
        pltpu.sync_copy(e_w, e_hbm.at[pl.ds(crow * B, CH0 * B)])

    @pl.when(c == 1)
    def _():
        pltpu.sync_copy(e_w.at[pl.ds(0, CH1 * B)],
                        e_hbm.at[pl.ds(crow * B, CH1 * B)])

    mx_v[...] = mx
    pltpu.sync_copy(mx_v, tmax_hbm.at[wid])


def _logits(xlp, xr, src2, dstg2, att):
    mesh = plsc.VectorSubcoreMesh(core_axis_name="c", subcore_axis_name="s",
                                  num_cores=NC, num_subcores=NS)
    return pl.kernel(
        _logits_body,
        out_type=(
            jax.ShapeDtypeStruct((EP,), jnp.float32),
            jax.ShapeDtypeStruct((NW, 16), jnp.float32),
        ),
        mesh=mesh,
        scratch_types=[
            pltpu.VMEM((C,), jnp.float32),
            pltpu.VMEM((CH0, B), jnp.int32),
            pltpu.VMEM((CH0, B), jnp.int32),
            pltpu.VMEM((CH0 * B,), jnp.float32),
            pltpu.VMEM((16,), jnp.float32),
            pltpu.VMEM((256,), jnp.float32),
            [pltpu.VMEM((B, CW), jnp.float32) for _ in range(2)],
            [pltpu.VMEM((B, C), jnp.float32) for _ in range(2)],
            [pltpu.SemaphoreType.DMA for _ in range(2)],
            [pltpu.SemaphoreType.DMA for _ in range(2)],
        ],
        compiler_params=_SC_PARAMS,
    )(xlp, xr, src2, dstg2, att)


def _aggregate_body(xlp_hbm, idxe_hbm, tmax_hbm, zeros_hbm,
                    accs_hbm,
                    idxe_v, dst_cur, w_v, tm_v, acc_sh,
                    rows_v, sem_i, sem_g, sem_s):
    c = lax.axis_index("c")
    s = lax.axis_index("s")
    crow = s * CH_SUM + c * CH0
    nch = jnp.where(c == 0, CH0, CH1)

    pltpu.sync_copy(zeros_hbm.at[pl.ds(s * RT, RT)],
                    acc_sh.at[pl.ds(s * RT, RT)])

    pltpu.sync_copy(tmax_hbm, tm_v)
    gmv = jnp.full((16,), -1e30, jnp.float32)
    for r in range(NW):
        gmv = jnp.maximum(gmv, tm_v[r])
    gmax = jnp.max(gmv)

    plsc.subcore_barrier()

    def idxe_copy(t, b):
        return pltpu.make_async_copy(idxe_hbm.at[crow + t], idxe_v.at[b],
                                     sem_i[b])

    def gather(t, b):
        return pltpu.make_async_copy(xlp_hbm.at[idxe_v.at[b, 0]], rows_v[b],
                                     sem_g[b])

    def scatter(b):
        return pltpu.make_async_copy(rows_v[b], acc_sh.at[dst_cur[b]],
                                     sem_s[b])

    idxe_copy(0, 0).start()
    idxe_copy(0, 0).wait()
    idxe_copy(1, 1).start()
    gather(0, 0).start()

    def pair(t2, carry):
        for b in range(2):
            t = t2 * 2 + b

            @pl.when(t >= 1)
            def _():
                scatter(1 - b).wait()

            @pl.when(t + 1 < nch)
            def _():
                idxe_copy(t + 1, 1 - b).wait()
                gather(t + 1, 1 - b).start()

            gather(t, b).wait()

            def pregroup(g, icarry):
                ds16 = pl.ds(g * 16, 16)
                ev = plsc.bitcast(idxe_v[b, 2, ds16], jnp.float32)
                w_v[ds16] = jnp.exp(ev - gmax)
                dst_cur[b][ds16] = idxe_v[b, 1, ds16]
                return icarry

            lax.fori_loop(0, B // 16, pregroup, 0)

            @pl.when(t + 2 < nch)
            def _():
                idxe_copy(t + 2, b).start()

            rv = rows_v[b]

            def edge(i, icarry):
                bw = plsc.load_gather(w_v, [jnp.full((16,), i, jnp.int32)])
                for j in range(CW // 16):
                    rv[i, pl.ds(j * 16, 16)] = rv[i, pl.ds(j * 16, 16)] * bw
                return icarry

            lax.fori_loop(0, B, edge, 0)
            scatter(b).start(add=True)
        return carry

    lax.fori_loop(0, nch // 2, pair, 0)
    scatter(1).wait()
    plsc.subcore_barrier()
    pltpu.sync_copy(acc_sh.at[pl.ds(s * RT, RT)],
                    accs_hbm.at[c, pl.ds(s * RT, RT)])


def _aggregate(xlp, idxe, tmax, zeros_acc):
    mesh = plsc.VectorSubcoreMesh(core_axis_name="c", subcore_axis_name="s",
                                  num_cores=NC, num_subcores=NS)
    return pl.kernel(
        _aggregate_body,
        out_type=jax.ShapeDtypeStruct((NC, NP, CW), jnp.float32),
        mesh=mesh,
        scratch_types=[
            pltpu.VMEM((2, 3, B), jnp.int32),
            [pltpu.VMEM((B,), jnp.int32) for _ in range(2)],
            pltpu.VMEM((B,), jnp.float32),
            pltpu.VMEM((NW, 16), jnp.float32),
            pltpu.VMEM_SHARED((NP, CW), jnp.float32),
            [pltpu.VMEM((B, CW), jnp.float32) for _ in range(2)],
            [pltpu.SemaphoreType.DMA for _ in range(2)],
            [pltpu.SemaphoreType.DMA for _ in range(2)],
            [pltpu.SemaphoreType.DMA for _ in range(2)],
        ],
        compiler_params=_SC_PARAMS,
    )(xlp, idxe, tmax, zeros_acc)


def _norm_body(accs_ref, bias_ref, gamma_ref, beta_ref, agn_ref, out_ref):
    a = accs_ref[0] + accs_ref[1]
    col = lax.broadcasted_iota(jnp.int32, (NP, CW), 1)
    den_full = jnp.where(col == C, a, 0.0)
    den = jnp.sum(den_full, axis=1, keepdims=True)
    num = a[:N, :C]
    out0 = num / (den[:N] + 1e-16) + bias_ref[...]
    mean = jnp.mean(out0, axis=0, keepdims=True)
    out_c = out0 - agn_ref[...] * mean
    var = jnp.mean(out_c * out_c, axis=0, keepdims=True)
    out_ref[...] = gamma_ref[...] * out_c / jnp.sqrt(var + 1e-5) + beta_ref[...]


def _finalize(accs, bias, gamma, beta, alpha_gn):
    return pl.pallas_call(
        _norm_body,
        out_shape=jax.ShapeDtypeStruct((N, C), jnp.float32),
    )(accs, bias.reshape(1, C), gamma.reshape(1, C), beta.reshape(1, C),
      alpha_gn.reshape(1, C))


@jax.jit
def kernel(x, edge_index, W_l, W_r, att, bias, gamma, beta, alpha_gn):
    ei = edge_index.astype(jnp.int32)
    loop = jnp.arange(N, dtype=jnp.int32)
    pad = EP - (E + N)
    src = jnp.concatenate([ei[0], loop, jnp.zeros((pad,), jnp.int32)])
    dstg = jnp.concatenate([ei[1], loop, jnp.zeros((pad,), jnp.int32)])
    dsts = jnp.concatenate([ei[1], loop, jnp.full((pad,), N, jnp.int32)])
    src2 = src.reshape(NW * CHUNKS, B)
    dstg2 = dstg.reshape(NW * CHUNKS, B)

    xl, xr = _project(x, W_l, W_r)
    extra = jnp.concatenate(
        [jnp.ones((N, 1), jnp.float32), jnp.zeros((N, 15), jnp.float32)], axis=1)
    xlp = jnp.concatenate([xl, extra], axis=1)

    e, tmax = _logits(xlp, xr, src2, dstg2, att.reshape(C))
    e_i32 = lax.bitcast_convert_type(e, jnp.int32)
    idxe = (jnp.stack([src, dsts, e_i32], axis=0)
            .reshape(3, NW * CHUNKS, B).transpose(1, 0, 2))
    zeros_acc = jnp.zeros((NP, CW), jnp.float32)
    accs = _aggregate(xlp, idxe, tmax, zeros_acc)
    return _finalize(accs, bias, gamma, beta, alpha_gn)
